# baseline (speedup 1.0000x reference)
import functools

import jax
import jax.numpy as jnp
from jax import lax
from jax.experimental import pallas as pl
from jax.experimental.pallas import tpu as pltpu
from jax.experimental.pallas import tpu_sc as plsc

_N = 10000
_NP = 10240
_E = 320000
_D = 128
_H = 64
_CH = 128
_TILES = 16
_RPT = _NP // _TILES
_NBUF = 5
_NCH = 160
_EP = _TILES * _NCH * _CH


def _tc_linear_body(lo_ref, hi_ref, wrel_ref, wroot_ref, b_ref,
                    xr_lo, xr_hi, xh_lo, xh_hi, *, relu):
    x = jnp.concatenate([lo_ref[...], hi_ref[...]], axis=1)
    if relu:
        x = jnp.maximum(x, 0.0)
    xr = jnp.dot(x, wrel_ref[...], preferred_element_type=jnp.float32)
    xr_lo[...] = xr[:, :_H]
    xr_hi[...] = xr[:, _H:]
    xh = jnp.dot(x, wroot_ref[...], preferred_element_type=jnp.float32) + b_ref[...]
    xh_lo[...] = xh[:, :_H]
    xh_hi[...] = xh[:, _H:]


def _tc_linear(lo, hi, w_rel, w_root, b, relu):
    blk = 1024
    grid = _NP // blk
    half_out = jax.ShapeDtypeStruct((_NP, _H), jnp.float32)
    return pl.pallas_call(
        functools.partial(_tc_linear_body, relu=relu),
        grid=(grid,),
        in_specs=[
            pl.BlockSpec((blk, _H), lambda i: (i, 0)),
            pl.BlockSpec((blk, _H), lambda i: (i, 0)),
            pl.BlockSpec((_D, _D), lambda i: (0, 0)),
            pl.BlockSpec((_D, _D), lambda i: (0, 0)),
            pl.BlockSpec((1, _D), lambda i: (0, 0)),
        ],
        out_specs=[pl.BlockSpec((blk, _H), lambda i: (i, 0))] * 4,
        out_shape=[half_out] * 4,
    )(lo, hi, w_rel, w_root, b.reshape(1, _D))


def _sc_body(xr_lo, xr_hi, xh_lo, xh_hi, src_hbm, dst_hbm,
             out_lo, out_hi, src_scr, dst_scr, rows_v, acc_sh,
             g0, g1, g2, g3, g4, s0, s1, s2, s3, s4):
    gsems = (g0, g1, g2, g3, g4)
    ssems = (s0, s1, s2, s3, s4)
    c = lax.axis_index("c")
    s = lax.axis_index("s")
    r0 = pl.multiple_of(s * _RPT, _RPT)
    crow = pl.multiple_of(s * _NCH, _NCH)

    def _half(xr, xh, out):
        pltpu.sync_copy(src_hbm.at[pl.ds(crow, _NCH)], src_scr)
        pltpu.sync_copy(dst_hbm.at[pl.ds(crow, _NCH)], dst_scr)
        pltpu.sync_copy(xh.at[pl.ds(r0, _RPT)], acc_sh.at[pl.ds(r0, _RPT)])
        plsc.subcore_barrier()

        def _scatter_desc(b):
            return pltpu.make_async_copy(
                rows_v.at[b], acc_sh.at[dst_scr.at[0]], ssems[b])

        def _group(g, carry):
            j0 = g * _NBUF
            for b in range(_NBUF):
                @pl.when(g > 0)
                def _():
                    _scatter_desc(b).wait()
                pltpu.async_copy(xr.at[src_scr.at[j0 + b]], rows_v.at[b],
                                 gsems[b])
            for b in range(_NBUF):
                pltpu.make_async_copy(xr.at[src_scr.at[0]], rows_v.at[b],
                                      gsems[b]).wait()
                pltpu.async_copy(rows_v.at[b], acc_sh.at[dst_scr.at[j0 + b]],
                                 ssems[b], add=True)
            return carry

        lax.fori_loop(0, _NCH // _NBUF, _group, 0)
        for b in range(_NBUF):
            _scatter_desc(b).wait()
        plsc.subcore_barrier()
        pltpu.sync_copy(acc_sh.at[pl.ds(r0, _RPT)], out.at[pl.ds(r0, _RPT)])

    @pl.when(c == 0)
    def _():
        _half(xr_lo, xh_lo, out_lo)

    @pl.when(c == 1)
    def _():
        _half(xr_hi, xh_hi, out_hi)


def _sc_aggregate(xr_lo, xr_hi, xh_lo, xh_hi, src2, dst2):
    half_out = jax.ShapeDtypeStruct((_NP, _H), jnp.float32)
    fn = pl.kernel(
        _sc_body,
        out_type=(half_out, half_out),
        mesh=plsc.VectorSubcoreMesh(core_axis_name="c", subcore_axis_name="s"),
        scratch_types=[
            pltpu.VMEM((_NCH, _CH), jnp.int32),
            pltpu.VMEM((_NCH, _CH), jnp.int32),
            pltpu.VMEM((_NBUF, _CH, _H), jnp.float32),
            pltpu.VMEM_SHARED((_NP, _H), jnp.float32),
        ] + [pltpu.SemaphoreType.DMA] * (2 * _NBUF),
        compiler_params=pltpu.CompilerParams(use_tc_tiling_on_sc=False),
    )
    return fn(xr_lo, xr_hi, xh_lo, xh_hi, src2, dst2)


def kernel(x, edge_index, W_rel0, b_rel0, W_root0, W_rel1, b_rel1, W_root1):
    epad = jnp.full((2, _EP - _E), _N, dtype=jnp.int32)
    eidx = jnp.concatenate([edge_index, epad], axis=1)
    src = eidx[0].reshape(_EP // _CH, _CH)
    dst = eidx[1].reshape(_EP // _CH, _CH)
    xp = jnp.pad(x, ((0, _NP - _N), (0, 0)))

    xr_lo, xr_hi, xh_lo, xh_hi = _tc_linear(
        xp[:, :_H], xp[:, _H:], W_rel0, W_root0, b_rel0, relu=False)
    s_lo, s_hi = _sc_aggregate(xr_lo, xr_hi, xh_lo, xh_hi, src, dst)

    hr_lo, hr_hi, hh_lo, hh_hi = _tc_linear(
        s_lo, s_hi, W_rel1, W_root1, b_rel1, relu=True)
    o_lo, o_hi = _sc_aggregate(hr_lo, hr_hi, hh_lo, hh_hi, src, dst)

    return jnp.concatenate([o_lo[:_N], o_hi[:_N]], axis=1)


# speedup vs baseline: 1.3841x; 1.3841x over previous
import functools

import jax
import jax.numpy as jnp
from jax import lax
from jax.experimental import pallas as pl
from jax.experimental.pallas import tpu as pltpu
from jax.experimental.pallas import tpu_sc as plsc

_N = 10000
_NP = 10240
_E = 320000
_D = 128
_H = 64
_CH = 128
_TILES = 16
_RPT = _NP // _TILES
_NBUF = 4
_NCH = 160
_EP = _TILES * _NCH * _CH


def _tc_linear_body(lo_ref, hi_ref, wrel_ref, wroot_ref, b_ref,
                    xr_lo, xr_hi, xh_lo, xh_hi, *, relu):
    x = jnp.concatenate([lo_ref[...], hi_ref[...]], axis=1)
    if relu:
        x = jnp.maximum(x, 0.0)
    xr = jnp.dot(x, wrel_ref[...], preferred_element_type=jnp.float32)
    xr_lo[...] = xr[:, :_H]
    xr_hi[...] = xr[:, _H:]
    xh = jnp.dot(x, wroot_ref[...], preferred_element_type=jnp.float32) + b_ref[...]
    xh_lo[...] = xh[:, :_H]
    xh_hi[...] = xh[:, _H:]


def _tc_linear(lo, hi, w_rel, w_root, b, relu):
    blk = 1024
    grid = _NP // blk
    half_out = jax.ShapeDtypeStruct((_NP, _H), jnp.float32)
    return pl.pallas_call(
        functools.partial(_tc_linear_body, relu=relu),
        grid=(grid,),
        in_specs=[
            pl.BlockSpec((blk, _H), lambda i: (i, 0)),
            pl.BlockSpec((blk, _H), lambda i: (i, 0)),
            pl.BlockSpec((_D, _D), lambda i: (0, 0)),
            pl.BlockSpec((_D, _D), lambda i: (0, 0)),
            pl.BlockSpec((1, _D), lambda i: (0, 0)),
        ],
        out_specs=[pl.BlockSpec((blk, _H), lambda i: (i, 0))] * 4,
        out_shape=[half_out] * 4,
    )(lo, hi, w_rel, w_root, b.reshape(1, _D))


def _sc_body(xr_lo, xr_hi, xh_lo, xh_hi, src_hbm, dst_hbm,
             out_lo, out_hi, srcb, dstb, rows_v, tab_sh, acc_sh,
             i0, i1, i2, i3, d0, d1, d2, d3,
             g0, g1, g2, g3, s0, s1, s2, s3):
    isems = (i0, i1, i2, i3)
    dsems = (d0, d1, d2, d3)
    gsems = (g0, g1, g2, g3)
    ssems = (s0, s1, s2, s3)
    c = lax.axis_index("c")
    s = lax.axis_index("s")
    r0 = pl.multiple_of(s * _RPT, _RPT)
    crow = pl.multiple_of(s * _NCH, _NCH)

    def _half(xr, xh, out):
        pltpu.sync_copy(xr.at[pl.ds(r0, _RPT)], tab_sh.at[pl.ds(r0, _RPT)])
        pltpu.sync_copy(xh.at[pl.ds(r0, _RPT)], acc_sh.at[pl.ds(r0, _RPT)])
        plsc.subcore_barrier()

        def _scatter_desc(b):
            return pltpu.make_async_copy(
                rows_v.at[b], acc_sh.at[dstb.at[0]], ssems[b])

        def _group(g, carry):
            j0 = g * _NBUF
            for b in range(_NBUF):
                @pl.when(g > 0)
                def _():
                    _scatter_desc(b).wait()
                pltpu.async_copy(src_hbm.at[crow + j0 + b], srcb.at[b],
                                 isems[b])
                pltpu.async_copy(dst_hbm.at[crow + j0 + b], dstb.at[b],
                                 dsems[b])
            for b in range(_NBUF):
                pltpu.make_async_copy(src_hbm.at[0], srcb.at[b],
                                      isems[b]).wait()
                pltpu.async_copy(tab_sh.at[srcb.at[b]], rows_v.at[b],
                                 gsems[b])
            for b in range(_NBUF):
                pltpu.make_async_copy(tab_sh.at[srcb.at[b]], rows_v.at[b],
                                      gsems[b]).wait()
                pltpu.make_async_copy(dst_hbm.at[0], dstb.at[b],
                                      dsems[b]).wait()
                pltpu.async_copy(rows_v.at[b], acc_sh.at[dstb.at[b]],
                                 ssems[b], add=True)
            return carry

        lax.fori_loop(0, _NCH // _NBUF, _group, 0)
        for b in range(_NBUF):
            _scatter_desc(b).wait()
        plsc.subcore_barrier()
        pltpu.sync_copy(acc_sh.at[pl.ds(r0, _RPT)], out.at[pl.ds(r0, _RPT)])

    @pl.when(c == 0)
    def _():
        _half(xr_lo, xh_lo, out_lo)

    @pl.when(c == 1)
    def _():
        _half(xr_hi, xh_hi, out_hi)


def _sc_aggregate(xr_lo, xr_hi, xh_lo, xh_hi, src2, dst2):
    half_out = jax.ShapeDtypeStruct((_NP, _H), jnp.float32)
    fn = pl.kernel(
        _sc_body,
        out_type=(half_out, half_out),
        mesh=plsc.VectorSubcoreMesh(core_axis_name="c", subcore_axis_name="s"),
        scratch_types=[
            pltpu.VMEM((_NBUF, _CH), jnp.int32),
            pltpu.VMEM((_NBUF, _CH), jnp.int32),
            pltpu.VMEM((_NBUF, _CH, _H), jnp.float32),
            pltpu.VMEM_SHARED((_NP, _H), jnp.float32),
            pltpu.VMEM_SHARED((_NP, _H), jnp.float32),
        ] + [pltpu.SemaphoreType.DMA] * (4 * _NBUF),
        compiler_params=pltpu.CompilerParams(use_tc_tiling_on_sc=False),
    )
    return fn(xr_lo, xr_hi, xh_lo, xh_hi, src2, dst2)


def kernel(x, edge_index, W_rel0, b_rel0, W_root0, W_rel1, b_rel1, W_root1):
    epad = jnp.full((2, _EP - _E), _N, dtype=jnp.int32)
    eidx = jnp.concatenate([edge_index, epad], axis=1)
    src = eidx[0].reshape(_EP // _CH, _CH)
    dst = eidx[1].reshape(_EP // _CH, _CH)
    xp = jnp.pad(x, ((0, _NP - _N), (0, 0)))

    xr_lo, xr_hi, xh_lo, xh_hi = _tc_linear(
        xp[:, :_H], xp[:, _H:], W_rel0, W_root0, b_rel0, relu=False)
    s_lo, s_hi = _sc_aggregate(xr_lo, xr_hi, xh_lo, xh_hi, src, dst)

    hr_lo, hr_hi, hh_lo, hh_hi = _tc_linear(
        s_lo, s_hi, W_rel1, W_root1, b_rel1, relu=True)
    o_lo, o_hi = _sc_aggregate(hr_lo, hr_hi, hh_lo, hh_hi, src, dst)

    return jnp.concatenate([o_lo[:_N], o_hi[:_N]], axis=1)


# speedup vs baseline: 1.7148x; 1.2389x over previous
import functools

import jax
import jax.numpy as jnp
from jax import lax
from jax.experimental import pallas as pl
from jax.experimental.pallas import tpu as pltpu
from jax.experimental.pallas import tpu_sc as plsc

_N = 10000
_NP = 10240
_E = 320000
_D = 128
_H = 64
_CH = 128
_TILES = 16
_RPT = _NP // _TILES
_NBUF = 3
_RPT2 = 400
_NCH = 160
_EP = _TILES * _NCH * _CH


def _tc_linear_body(lo_ref, hi_ref, wrel_ref, wroot_ref, b_ref,
                    xr_lo, xr_hi, xh_lo, xh_hi, *, relu):
    x = jnp.concatenate([lo_ref[...], hi_ref[...]], axis=1)
    if relu:
        x = jnp.maximum(x, 0.0)
    xr = jnp.dot(x, wrel_ref[...], preferred_element_type=jnp.float32)
    xr_lo[...] = xr[:, :_H]
    xr_hi[...] = xr[:, _H:]
    xh = jnp.dot(x, wroot_ref[...], preferred_element_type=jnp.float32) + b_ref[...]
    xh_lo[...] = xh[:, :_H]
    xh_hi[...] = xh[:, _H:]


def _tc_linear(lo, hi, w_rel, w_root, b, relu):
    blk = 1024
    grid = _NP // blk
    half_out = jax.ShapeDtypeStruct((_NP, _H), jnp.float32)
    return pl.pallas_call(
        functools.partial(_tc_linear_body, relu=relu),
        grid=(grid,),
        in_specs=[
            pl.BlockSpec((blk, _H), lambda i: (i, 0)),
            pl.BlockSpec((blk, _H), lambda i: (i, 0)),
            pl.BlockSpec((_D, _D), lambda i: (0, 0)),
            pl.BlockSpec((_D, _D), lambda i: (0, 0)),
            pl.BlockSpec((1, _D), lambda i: (0, 0)),
        ],
        out_specs=[pl.BlockSpec((blk, _H), lambda i: (i, 0))] * 4,
        out_shape=[half_out] * 4,
    )(lo, hi, w_rel, w_root, b.reshape(1, _D))


def _sc_body(xr_lo, xr_hi, xh_lo, xh_hi, src_hbm, dst_hbm, *refs, full):
    if full:
        (out, src_scr, dstb, rows_v, tab_sh, acc_sh,
         d0, d1, d2, g0, g1, g2, s0, s1, s2) = refs
        out_lo = out_hi = out
    else:
        (out_lo, out_hi, src_scr, dstb, rows_v, tab_sh, acc_sh,
         d0, d1, d2, g0, g1, g2, s0, s1, s2) = refs
    dsems = (d0, d1, d2)
    gsems = (g0, g1, g2)
    ssems = (s0, s1, s2)
    c = lax.axis_index("c")
    s = lax.axis_index("s")
    r0 = pl.multiple_of(s * _RPT, _RPT)
    crow = pl.multiple_of(s * _NCH, _NCH)

    def _drain(out, col):
        if not full:
            pltpu.sync_copy(acc_sh.at[pl.ds(r0, _RPT)], out.at[pl.ds(r0, _RPT)])
            return
        @pl.when(s < _TILES - 1)
        def _():
            pltpu.sync_copy(acc_sh.at[pl.ds(r0, _RPT)],
                            out.at[pl.ds(r0, _RPT), pl.ds(col, _H)])
        @pl.when(s == _TILES - 1)
        def _():
            pltpu.sync_copy(acc_sh.at[pl.ds(_N - _RPT2, _RPT2)],
                            out.at[pl.ds(_N - _RPT2, _RPT2), pl.ds(col, _H)])

    def _half(xr, xh, out, col):
        pltpu.sync_copy(src_hbm.at[pl.ds(crow, _NCH)], src_scr)
        pltpu.sync_copy(xr.at[pl.ds(r0, _RPT)], tab_sh.at[pl.ds(r0, _RPT)])
        pltpu.sync_copy(xh.at[pl.ds(r0, _RPT)], acc_sh.at[pl.ds(r0, _RPT)])
        plsc.subcore_barrier()

        def _scatter_desc(b):
            return pltpu.make_async_copy(
                rows_v.at[b], acc_sh.at[dstb.at[0]], ssems[b])

        def _group(g, carry):
            j0 = g * _NBUF
            for b in range(_NBUF):
                @pl.when(g > 0)
                def _():
                    _scatter_desc(b).wait()
                pltpu.async_copy(dst_hbm.at[crow + j0 + b], dstb.at[b],
                                 dsems[b])
                pltpu.async_copy(tab_sh.at[src_scr.at[j0 + b]], rows_v.at[b],
                                 gsems[b])
            for b in range(_NBUF):
                pltpu.make_async_copy(tab_sh.at[src_scr.at[0]], rows_v.at[b],
                                      gsems[b]).wait()
                pltpu.make_async_copy(dst_hbm.at[0], dstb.at[b],
                                      dsems[b]).wait()
                pltpu.async_copy(rows_v.at[b], acc_sh.at[dstb.at[b]],
                                 ssems[b], add=True)
            return carry

        lax.fori_loop(0, _NCH // _NBUF, _group, 0)
        for b in range(_NBUF):
            _scatter_desc(b).wait()
        plsc.subcore_barrier()
        _drain(out, col)

    @pl.when(c == 0)
    def _():
        _half(xr_lo, xh_lo, out_lo, 0)

    @pl.when(c == 1)
    def _():
        _half(xr_hi, xh_hi, out_hi, _H)


def _sc_aggregate(xr_lo, xr_hi, xh_lo, xh_hi, src2, dst2, full):
    half_out = jax.ShapeDtypeStruct((_NP, _H), jnp.float32)
    if full:
        outs = jax.ShapeDtypeStruct((_N, _D), jnp.float32)
    else:
        outs = (half_out, half_out)
    fn = pl.kernel(
        functools.partial(_sc_body, full=full),
        out_type=outs,
        mesh=plsc.VectorSubcoreMesh(core_axis_name="c", subcore_axis_name="s"),
        scratch_types=[
            pltpu.VMEM((_NCH, _CH), jnp.int32),
            pltpu.VMEM((_NBUF, _CH), jnp.int32),
            pltpu.VMEM((_NBUF, _CH, _H), jnp.float32),
            pltpu.VMEM_SHARED((_NP, _H), jnp.float32),
            pltpu.VMEM_SHARED((_NP, _H), jnp.float32),
        ] + [pltpu.SemaphoreType.DMA] * (3 * _NBUF),
        compiler_params=pltpu.CompilerParams(use_tc_tiling_on_sc=False),
    )
    return fn(xr_lo, xr_hi, xh_lo, xh_hi, src2, dst2)


def kernel(x, edge_index, W_rel0, b_rel0, W_root0, W_rel1, b_rel1, W_root1):
    epad = jnp.full((2, _EP - _E), _N, dtype=jnp.int32)
    eidx = jnp.concatenate([edge_index, epad], axis=1)
    src = eidx[0].reshape(_EP // _CH, _CH)
    dst = eidx[1].reshape(_EP // _CH, _CH)
    xp = jnp.pad(x, ((0, _NP - _N), (0, 0)))

    xr_lo, xr_hi, xh_lo, xh_hi = _tc_linear(
        xp[:, :_H], xp[:, _H:], W_rel0, W_root0, b_rel0, relu=False)
    s_lo, s_hi = _sc_aggregate(xr_lo, xr_hi, xh_lo, xh_hi, src, dst, full=False)

    hr_lo, hr_hi, hh_lo, hh_hi = _tc_linear(
        s_lo, s_hi, W_rel1, W_root1, b_rel1, relu=True)
    return _sc_aggregate(hr_lo, hr_hi, hh_lo, hh_hi, src, dst, full=True)
